# baseline (speedup 1.0000x reference)
import jax
import jax.numpy as jnp
from jax import lax
from jax.experimental import pallas as pl
from jax.experimental.pallas import tpu as pltpu
from jax.experimental.pallas import tpu_sc as plsc

BATCH = 4096
N_CDD = 50
DIM = 128
L = 16
NC = 2
NS = 16
NW = NC * NS
U_PER_W = BATCH // NW
U_CHUNK = 8
N_CHUNKS = U_PER_W // U_CHUNK
KPAD = 64
NBUF = 2


def _body(uid_hbm, cdd_hbm, news_hbm, user_hbm, out_hbm,
          uid_v, cdd_v, urows_v, nrows0, nrows1, sc0, sc1,
          usem, gsem0, gsem1, wsem0, wsem1):
    wid = lax.axis_index("s") * NC + lax.axis_index("c")
    lane = lax.iota(jnp.int32, L)
    nrows = (nrows0, nrows1)
    scores = (sc0, sc1)
    gsem = (gsem0, gsem1)
    wsem = (wsem0, wsem1)
    wbase = wid * U_PER_W

    pltpu.sync_copy(uid_hbm.at[pl.ds(wbase, U_PER_W)], uid_v)
    pltpu.sync_copy(cdd_hbm.at[pl.ds(wbase * N_CDD, U_PER_W * N_CDD)], cdd_v)
    pltpu.async_copy(user_hbm.at[uid_v], urows_v, usem).wait()

    def gathers(b, c):
        return [pltpu.make_async_copy(
            news_hbm.at[cdd_v.at[pl.ds(c * U_CHUNK * N_CDD, U_CHUNK * N_CDD)]],
            nrows[b], gsem[b])]

    def issue(b, c):
        for cp in gathers(b, c):
            cp.start()

    def drain(b, c):
        for cp in gathers(b, c):
            cp.wait()

    def compute(b, c):
        @pl.when(c >= NBUF)
        def _():
            pltpu.make_async_copy(
                scores[b], out_hbm.at[pl.ds(0, U_CHUNK), :], wsem[b]).wait()

        def user_body(u, _):
            uvec = [urows_v[c * U_CHUNK + u, pl.ds(d * L, L)]
                    for d in range(DIM // L)]
            row0 = u * N_CDD
            for gp in range(2):
                svecs = []
                for g in (2 * gp, 2 * gp + 1):
                    svec = jnp.zeros((L,), jnp.float32)
                    for kk in range(L):
                        k = g * L + kk
                        if k >= N_CDD:
                            break
                        acc = nrows[b][row0 + k, pl.ds(0, L)] * uvec[0]
                        for d in range(1, DIM // L):
                            acc += nrows[b][row0 + k, pl.ds(d * L, L)] * uvec[d]
                        s = plsc.cumsum(acc)[L - 1]
                        svec = jnp.where(lane == kk, s, svec)
                    svecs.append(svec)
                for i, g in enumerate((2 * gp, 2 * gp + 1)):
                    scores[b][u, pl.ds(g * L, L)] = (
                        1.0 / (1.0 + jnp.exp(-svecs[i])))
            return 0

        lax.fori_loop(0, U_CHUNK, user_body, 0)
        pltpu.async_copy(
            scores[b], out_hbm.at[pl.ds(wbase + c * U_CHUNK, U_CHUNK), :],
            wsem[b])

    issue(0, 0)

    def outer(i, _):
        for b in range(NBUF):
            c = NBUF * i + b
            nb = (b + 1) % NBUF

            @pl.when(c + 1 < N_CHUNKS)
            def _():
                issue(nb, c + 1)

            drain(b, c)
            compute(b, c)
        return 0

    lax.fori_loop(0, N_CHUNKS // NBUF, outer, 0)

    for b in range(NBUF):
        pltpu.make_async_copy(
            scores[b], out_hbm.at[pl.ds(0, U_CHUNK), :], wsem[b]).wait()


def kernel(user_id, cdd_id, news_table, user_table):
    mesh = plsc.VectorSubcoreMesh(
        core_axis_name="c", subcore_axis_name="s",
        num_cores=NC, num_subcores=NS)
    k = pl.kernel(
        _body,
        out_type=jax.ShapeDtypeStruct((BATCH, KPAD), jnp.float32),
        mesh=mesh,
        compiler_params=pltpu.CompilerParams(
            needs_layout_passes=False, use_tc_tiling_on_sc=False),
        scratch_types=[
            pltpu.VMEM((U_PER_W,), jnp.int32),
            pltpu.VMEM((U_PER_W * N_CDD,), jnp.int32),
            pltpu.VMEM((U_PER_W, DIM), jnp.float32),
            pltpu.VMEM((U_CHUNK * N_CDD, DIM), jnp.float32),
            pltpu.VMEM((U_CHUNK * N_CDD, DIM), jnp.float32),
            pltpu.VMEM((U_CHUNK, KPAD), jnp.float32),
            pltpu.VMEM((U_CHUNK, KPAD), jnp.float32),
            pltpu.SemaphoreType.DMA,
            pltpu.SemaphoreType.DMA,
            pltpu.SemaphoreType.DMA,
            pltpu.SemaphoreType.DMA,
            pltpu.SemaphoreType.DMA,
        ],
    )
    return k(user_id, cdd_id.reshape(-1), news_table, user_table)[:, :N_CDD]


# speedup vs baseline: 1.0120x; 1.0120x over previous
import jax
import jax.numpy as jnp
from jax import lax
from jax.experimental import pallas as pl
from jax.experimental.pallas import tpu as pltpu
from jax.experimental.pallas import tpu_sc as plsc

BATCH = 4096
N_CDD = 50
DIM = 128
L = 16
NC = 2
NS = 16
NW = NC * NS
U_PER_W = BATCH // NW
U_CHUNK = 8
N_CHUNKS = U_PER_W // U_CHUNK
KPAD = 64
NBUF = 2


def _body(uid_hbm, cdd_hbm, news_hbm, user_hbm, out_hbm,
          uid_v, cdd_v, urows0, urows1, nrows0, nrows1, sc0, sc1,
          gsem0, gsem1, wsem0, wsem1):
    wid = lax.axis_index("s") * NC + lax.axis_index("c")
    lane = lax.iota(jnp.int32, L)
    urows = (urows0, urows1)
    nrows = (nrows0, nrows1)
    scores = (sc0, sc1)
    gsem = (gsem0, gsem1)
    wsem = (wsem0, wsem1)
    wbase = wid * U_PER_W

    pltpu.sync_copy(uid_hbm.at[pl.ds(wbase, U_PER_W)], uid_v)
    pltpu.sync_copy(cdd_hbm.at[pl.ds(wbase, U_PER_W), :], cdd_v)

    def gathers(b, c):
        cps = [pltpu.make_async_copy(
            user_hbm.at[uid_v.at[pl.ds(c * U_CHUNK, U_CHUNK)]],
            urows[b], gsem[b])]
        for u in range(U_CHUNK):
            cps.append(pltpu.make_async_copy(
                news_hbm.at[cdd_v.at[c * U_CHUNK + u]],
                nrows[b].at[pl.ds(u * N_CDD, N_CDD), :], gsem[b]))
        return cps

    def issue(b, c):
        for cp in gathers(b, c):
            cp.start()

    def drain(b, c):
        for cp in gathers(b, c):
            cp.wait()

    def compute(b, c):
        @pl.when(c >= NBUF)
        def _():
            pltpu.make_async_copy(
                scores[b], out_hbm.at[pl.ds(0, U_CHUNK), :], wsem[b]).wait()

        def user_body(u, _):
            uvec = [urows[b][u, pl.ds(d * L, L)] for d in range(DIM // L)]
            row0 = u * N_CDD
            for gp in range(2):
                svecs = []
                for g in (2 * gp, 2 * gp + 1):
                    svec = jnp.zeros((L,), jnp.float32)
                    for kk in range(L):
                        k = g * L + kk
                        if k >= N_CDD:
                            break
                        acc = nrows[b][row0 + k, pl.ds(0, L)] * uvec[0]
                        for d in range(1, DIM // L):
                            acc += nrows[b][row0 + k, pl.ds(d * L, L)] * uvec[d]
                        s = plsc.cumsum(acc)[L - 1]
                        svec = jnp.where(lane == kk, s, svec)
                    svecs.append(svec)
                for i, g in enumerate((2 * gp, 2 * gp + 1)):
                    scores[b][u, pl.ds(g * L, L)] = (
                        1.0 / (1.0 + jnp.exp(-svecs[i])))
            return 0

        lax.fori_loop(0, U_CHUNK, user_body, 0)
        pltpu.async_copy(
            scores[b], out_hbm.at[pl.ds(wbase + c * U_CHUNK, U_CHUNK), :],
            wsem[b])

    issue(0, 0)

    def outer(i, _):
        for b in range(NBUF):
            c = NBUF * i + b
            nb = (b + 1) % NBUF

            @pl.when(c + 1 < N_CHUNKS)
            def _():
                issue(nb, c + 1)

            drain(b, c)
            compute(b, c)
        return 0

    lax.fori_loop(0, N_CHUNKS // NBUF, outer, 0)

    for b in range(NBUF):
        pltpu.make_async_copy(
            scores[b], out_hbm.at[pl.ds(0, U_CHUNK), :], wsem[b]).wait()


def kernel(user_id, cdd_id, news_table, user_table):
    mesh = plsc.VectorSubcoreMesh(
        core_axis_name="c", subcore_axis_name="s",
        num_cores=NC, num_subcores=NS)
    k = pl.kernel(
        _body,
        out_type=jax.ShapeDtypeStruct((BATCH, KPAD), jnp.float32),
        mesh=mesh,
        compiler_params=pltpu.CompilerParams(
            needs_layout_passes=False, use_tc_tiling_on_sc=False),
        scratch_types=[
            pltpu.VMEM((U_PER_W,), jnp.int32),
            pltpu.VMEM((U_PER_W, N_CDD), jnp.int32),
            pltpu.VMEM((U_CHUNK, DIM), jnp.float32),
            pltpu.VMEM((U_CHUNK, DIM), jnp.float32),
            pltpu.VMEM((U_CHUNK * N_CDD, DIM), jnp.float32),
            pltpu.VMEM((U_CHUNK * N_CDD, DIM), jnp.float32),
            pltpu.VMEM((U_CHUNK, KPAD), jnp.float32),
            pltpu.VMEM((U_CHUNK, KPAD), jnp.float32),
            pltpu.SemaphoreType.DMA,
            pltpu.SemaphoreType.DMA,
            pltpu.SemaphoreType.DMA,
            pltpu.SemaphoreType.DMA,
        ],
    )
    return k(user_id, cdd_id, news_table, user_table)[:, :N_CDD]
